# baseline (speedup 1.0000x reference)
import functools

import jax
import jax.numpy as jnp
from jax import lax
from jax.experimental import pallas as pl
from jax.experimental.pallas import tpu as pltpu
from jax.experimental.pallas import tpu_sc as plsc

NX = 13
NODES = NX * NX
NPAD = 256
L_SIZE = 0.1
THICKNESS = 0.001
RHO = 2700.0
CP = 900.0
DT = 1.0
DX = L_SIZE / (NX - 1)
DY = L_SIZE / (NX - 1)
BOLTZ = 5.67e-08
VOL_A = RHO * CP * THICKNESS * DX * DY / DT

NW = 32
ROWS = 32
SA = 128
SB = NODES - SA
LASTB = 153


def _coef_table(K, E):
    z1 = jnp.zeros((1,), jnp.float32)
    z13 = jnp.zeros((13,), jnp.float32)
    kc = jnp.diagonal(K) + VOL_A * (-1.0)
    kl = jnp.concatenate([z1, jnp.diagonal(K, -1)])
    kr = jnp.concatenate([jnp.diagonal(K, 1), z1])
    kd = jnp.concatenate([z13, jnp.diagonal(K, -13)])
    ku = jnp.concatenate([jnp.diagonal(K, 13), z13])
    be = BOLTZ * jnp.diagonal(E)
    full = jnp.stack([jnp.asarray(r, jnp.float32)
                      for r in (kc, kl, kr, kd, ku, be)])
    tab = jnp.zeros((8, NPAD), jnp.float32)
    tab = tab.at[:6, :160].set(full[:, :160])
    tab = tab.at[:6, 160:176].set(full[:, LASTB:NODES])
    mask10 = jnp.concatenate([jnp.zeros((160 - LASTB,), jnp.float32),
                              jnp.ones((NODES - 160,), jnp.float32)])
    tab = tab.at[6, 160:176].set(mask10)
    return tab


@functools.lru_cache(maxsize=None)
def _build_sc(B):
    assert B % (NW * ROWS * 2) == 0
    chunks = B // (NW * ROWS)
    pairs = chunks // 2
    mesh = plsc.VectorSubcoreMesh(core_axis_name="c", subcore_axis_name="s")

    @functools.partial(
        pl.kernel,
        mesh=mesh,
        out_type=jax.ShapeDtypeStruct((NW, 16), jnp.float32),
        compiler_params=pltpu.CompilerParams(use_tc_tiling_on_sc=True,
                                             needs_layout_passes=False),
        scratch_types=(
            [pltpu.VMEM((ROWS, NODES), jnp.float32) for _ in range(10)]
            + [
                pltpu.VMEM((8, NPAD), jnp.float32),
                pltpu.VMEM((16,), jnp.float32),
                pltpu.SemaphoreType.DMA,
                pltpu.SemaphoreType.DMA,
            ]
        ),
    )
    def sc_loss(tn_h, to_h, ht_h, if_h, te_h, coef_h, out_h,
                x0, x1, x2, x3, x4, y0, y1, y2, y3, y4,
                cf, accv, semA, semB):
        wid = lax.axis_index("s") * 2 + lax.axis_index("c")
        srcs = (to_h, tn_h, ht_h, if_h, te_h)
        halves = ((x0, x1, x2, x3, x4), (y0, y1, y2, y3, y4))
        pltpu.sync_copy(coef_h, cf)
        lane = lax.iota(jnp.int32, 16)
        gc_l0 = jnp.maximum(lane - 1, 0)
        gc_d0 = jnp.maximum(lane - 13, 0)
        gc_r7 = 113 + lane
        gc_u7 = 125 + lane
        gc_l8 = 127 + lane
        gc_d8 = 115 + lane
        gc_u9 = jnp.minimum(157 + lane, NODES - 1)
        gc_r10 = jnp.minimum(LASTB + 1 + lane, NODES - 1)

        def copies(half, r0):
            out = []
            for src, dst in zip(srcs, halves[half]):
                out.append((src.at[pl.ds(r0, ROWS), pl.ds(0, SA)],
                            dst.at[pl.ds(0, ROWS), pl.ds(0, SA)]))
                out.append((src.at[pl.ds(r0, ROWS), pl.ds(SA, SB)],
                            dst.at[pl.ds(0, ROWS), pl.ds(SA, SB)]))
            return out

        def issue(half, r0, sem):
            for s, d in copies(half, r0):
                pltpu.make_async_copy(s, d, sem).start()

        def drain(half, r0, sem):
            for s, d in copies(half, r0):
                pltpu.make_async_copy(s, d, sem).wait()

        def compute(half, acc):
            gTo, gTn, gHt, gIf, gTe = halves[half]
            for vb in range(11):
                cc = 16 * vb
                nb = LASTB if vb == 10 else 16 * vb
                kc = cf[0, pl.ds(cc, 16)]
                kl = cf[1, pl.ds(cc, 16)]
                kr = cf[2, pl.ds(cc, 16)]
                kd = cf[3, pl.ds(cc, 16)]
                ku = cf[4, pl.ds(cc, 16)]
                be = cf[5, pl.ds(cc, 16)]
                m10 = cf[6, pl.ds(cc, 16)]

                def row_body(g, a, _vb=vb, _nb=nb, _kc=kc, _kl=kl, _kr=kr,
                             _kd=kd, _ku=ku, _be=be, _m=m10):
                    for u in range(4):
                        r = g * 4 + u
                        rv = jnp.full((16,), r, dtype=jnp.int32)
                        toc = gTo[r, pl.ds(_nb, 16)]
                        if _vb == 0:
                            tol = plsc.load_gather(gTo, [rv, gc_l0])
                            tod = plsc.load_gather(gTo, [rv, gc_d0])
                        elif _vb == 8:
                            tol = plsc.load_gather(gTo, [rv, gc_l8])
                            tod = plsc.load_gather(gTo, [rv, gc_d8])
                        else:
                            tol = gTo[r, pl.ds(_nb - 1, 16)]
                            tod = gTo[r, pl.ds(_nb - 13, 16)]
                        if _vb == 7:
                            tor = plsc.load_gather(gTo, [rv, gc_r7])
                            tou = plsc.load_gather(gTo, [rv, gc_u7])
                        elif _vb == 9:
                            tor = gTo[r, pl.ds(_nb + 1, 16)]
                            tou = plsc.load_gather(gTo, [rv, gc_u9])
                        elif _vb == 10:
                            tor = plsc.load_gather(gTo, [rv, gc_r10])
                            tou = None
                        else:
                            tor = gTo[r, pl.ds(_nb + 1, 16)]
                            tou = gTo[r, pl.ds(_nb + 13, 16)]
                        tnn = gTn[r, pl.ds(_nb, 16)]
                        q1 = gHt[r, pl.ds(_nb, 16)]
                        q2 = gIf[r, pl.ds(_nb, 16)]
                        tee = gTe[r, pl.ds(_nb, 16)]
                        t = VOL_A * tnn + _kc * toc
                        t = t + _kl * tol + _kr * tor
                        t = t + _kd * tod
                        if tou is not None:
                            t = t + _ku * tou
                        t = t - q1 - q2
                        to2 = toc * toc
                        te2 = tee * tee
                        t = t + _be * (to2 * to2 - te2 * te2)
                        t = jnp.abs(t)
                        if _vb == 10:
                            t = _m * t
                        a = a + t
                    return a

                acc = lax.fori_loop(0, ROWS // 4, row_body, acc)
            return acc

        base = wid * (chunks * ROWS)
        issue(0, base, semA)

        def pair_body(t, acc):
            offA = base + (2 * t) * ROWS
            offB = offA + ROWS
            issue(1, offB, semB)
            drain(0, offA, semA)
            acc = compute(0, acc)
            nextA = offB + ROWS

            @pl.when(t + 1 < pairs)
            def _():
                issue(0, nextA, semA)

            drain(1, offB, semB)
            acc = compute(1, acc)
            return acc

        acc = lax.fori_loop(0, pairs, pair_body,
                            jnp.zeros((16,), jnp.float32))
        accv[...] = acc
        pltpu.sync_copy(accv, out_h.at[wid])

    return sc_loss


def kernel(T_new, T_old, heaters_input, interfaces_input, Tenv, K, E):
    B = T_new.shape[0]
    coef = _coef_table(K, E)
    sc = _build_sc(B)
    partials = sc(T_new, T_old,
                  heaters_input.astype(jnp.float32),
                  interfaces_input.astype(jnp.float32),
                  Tenv, coef)
    return jnp.sum(partials) / (B * NODES)


# speedup vs baseline: 2.0522x; 2.0522x over previous
import functools

import jax
import jax.numpy as jnp
from jax import lax
from jax.experimental import pallas as pl
from jax.experimental.pallas import tpu as pltpu
from jax.experimental.pallas import tpu_sc as plsc

NX = 13
NODES = NX * NX
L_SIZE = 0.1
THICKNESS = 0.001
RHO = 2700.0
CP = 900.0
DT = 1.0
DX = L_SIZE / (NX - 1)
DY = L_SIZE / (NX - 1)
BOLTZ = 5.67e-08
VOL_A = RHO * CP * THICKNESS * DX * DY / DT

NW = 32
CT = 128
NCT = 4
N0S = (0, 56, 112)
N1S = (56, 112, NODES)
H0S = (0, 40, 96)
HRS = (72, 88, NODES - 96)
TO_ROWS = 88
X_ROWS = 64


def _coef_table(K, E):
    z1 = jnp.zeros((1,), jnp.float32)
    z13 = jnp.zeros((13,), jnp.float32)
    kc = jnp.diagonal(K) + VOL_A * (-1.0)
    kl = jnp.concatenate([z1, jnp.diagonal(K, -1)])
    kr = jnp.concatenate([jnp.diagonal(K, 1), z1])
    kd = jnp.concatenate([z13, jnp.diagonal(K, -13)])
    ku = jnp.concatenate([jnp.diagonal(K, 13), z13])
    be = BOLTZ * jnp.diagonal(E)
    t = jnp.stack([jnp.asarray(r, jnp.float32)
                   for r in (kc, kl, kr, kd, ku, be)], axis=1)
    return jnp.repeat(t, 16, axis=1)


@functools.lru_cache(maxsize=None)
def _build_sc(B):
    assert B == NW * NCT * CT
    mesh = plsc.VectorSubcoreMesh(core_axis_name="c", subcore_axis_name="s")
    units = [(ct, nh) for ct in range(NCT) for nh in range(3)]

    @functools.partial(
        pl.kernel,
        mesh=mesh,
        out_type=jax.ShapeDtypeStruct((NW, 16), jnp.float32),
        compiler_params=pltpu.CompilerParams(use_tc_tiling_on_sc=True),
        scratch_types=(
            [pltpu.VMEM((TO_ROWS, CT), jnp.float32),
             pltpu.VMEM((X_ROWS, CT), jnp.float32),
             pltpu.VMEM((X_ROWS, CT), jnp.float32),
             pltpu.VMEM((X_ROWS, CT), jnp.float32),
             pltpu.VMEM((X_ROWS, CT), jnp.float32)] * 2
            + [
                pltpu.VMEM((NODES, 96), jnp.float32),
                pltpu.VMEM((16,), jnp.float32),
                pltpu.SemaphoreType.DMA,
                pltpu.SemaphoreType.DMA,
            ]
        ),
    )
    def sc_loss(to_h, tn_h, ht_h, if_h, te_h, coef_h, out_h,
                x0, x1, x2, x3, x4, y0, y1, y2, y3, y4,
                cf, accv, semA, semB):
        wid = lax.axis_index("s") * 2 + lax.axis_index("c")
        srcs = (to_h, tn_h, ht_h, if_h, te_h)
        halves = ((x0, x1, x2, x3, x4), (y0, y1, y2, y3, y4))
        sems = (semA, semB)
        pltpu.sync_copy(coef_h, cf)
        colbase = wid * (NCT * CT)

        def copies(u):
            ct, nh = units[u]
            bufs = halves[u % 2]
            c0 = colbase + ct * CT
            out = [(srcs[0].at[pl.ds(H0S[nh], HRS[nh]), pl.ds(c0, CT)],
                    bufs[0].at[pl.ds(0, HRS[nh]), pl.ds(0, CT)])]
            nr = N1S[nh] - N0S[nh]
            for k in range(1, 5):
                out.append((srcs[k].at[pl.ds(N0S[nh], nr), pl.ds(c0, CT)],
                            bufs[k].at[pl.ds(0, nr), pl.ds(0, CT)]))
            return out

        def issue(u):
            for s, d in copies(u):
                pltpu.make_async_copy(s, d, sems[u % 2]).start()

        def drain(u):
            for s, d in copies(u):
                pltpu.make_async_copy(s, d, sems[u % 2]).wait()

        def compute(u, acc):
            ct, nh = units[u]
            gTo, gTn, gHt, gIf, gTe = halves[u % 2]
            n0, n1, h0 = N0S[nh], N1S[nh], H0S[nh]
            nr = n1 - n0
            hr = HRS[nh]
            dT = n0 - h0

            def node_body(i, a, _n0=n0, _dT=dT, _hr=hr, _nh=nh):
                kc = cf[i + _n0, pl.ds(0, 16)]
                kl = cf[i + _n0, pl.ds(16, 16)]
                kr = cf[i + _n0, pl.ds(32, 16)]
                kd = cf[i + _n0, pl.ds(48, 16)]
                ku = cf[i + _n0, pl.ds(64, 16)]
                be = cf[i + _n0, pl.ds(80, 16)]
                rT = i + _dT
                rl = jnp.maximum(rT - 1, 0) if _nh == 0 else rT - 1
                rd = jnp.maximum(rT - 13, 0) if _nh == 0 else rT - 13
                rr = jnp.minimum(rT + 1, _hr - 1) if _nh == 2 else rT + 1
                ru = jnp.minimum(rT + 13, _hr - 1) if _nh == 2 else rT + 13
                for j in range(CT // 16):
                    cb = 16 * j
                    toc = gTo[rT, pl.ds(cb, 16)]
                    tol = gTo[rl, pl.ds(cb, 16)]
                    tor = gTo[rr, pl.ds(cb, 16)]
                    tod = gTo[rd, pl.ds(cb, 16)]
                    tou = gTo[ru, pl.ds(cb, 16)]
                    tnn = gTn[i, pl.ds(cb, 16)]
                    q1 = gHt[i, pl.ds(cb, 16)]
                    q2 = gIf[i, pl.ds(cb, 16)]
                    tee = gTe[i, pl.ds(cb, 16)]
                    t = VOL_A * tnn + kc * toc
                    t = t + kl * tol + kr * tor
                    t = t + kd * tod + ku * tou
                    t = t - q1 - q2
                    to2 = toc * toc
                    te2 = tee * tee
                    t = t + be * (to2 * to2 - te2 * te2)
                    a = a + jnp.abs(t)
                return a

            return lax.fori_loop(0, nr, node_body, acc)

        acc = jnp.zeros((16,), jnp.float32)
        issue(0)
        issue(1)
        for u in range(len(units)):
            drain(u)
            if u + 2 < len(units):
                issue(u + 2)
            acc = compute(u, acc)
        accv[...] = acc
        pltpu.sync_copy(accv, out_h.at[wid])

    return sc_loss


def kernel(T_new, T_old, heaters_input, interfaces_input, Tenv, K, E):
    B = T_new.shape[0]
    coef = _coef_table(K, E)
    sc = _build_sc(B)
    partials = sc(T_old.T, T_new.T,
                  heaters_input.astype(jnp.float32).T,
                  interfaces_input.astype(jnp.float32).T,
                  Tenv.T, coef)
    return jnp.sum(partials) / (B * NODES)


# speedup vs baseline: 2.4353x; 1.1867x over previous
import functools

import jax
import jax.numpy as jnp
from jax import lax
from jax.experimental import pallas as pl
from jax.experimental.pallas import tpu as pltpu
from jax.experimental.pallas import tpu_sc as plsc

NX = 13
NODES = NX * NX
L_SIZE = 0.1
THICKNESS = 0.001
RHO = 2700.0
CP = 900.0
DT = 1.0
DX = L_SIZE / (NX - 1)
DY = L_SIZE / (NX - 1)
BOLTZ = 5.67e-08
VOL_A = RHO * CP * THICKNESS * DX * DY / DT

NW = 32
CT = 128
NCT = 4
N0S = (0, 56, 112)
N1S = (56, 112, NODES)
H0S = (0, 40, 96)
HRS = (72, 88, NODES - 96)
TO_ROWS = 88
X_ROWS = 64


def _coef_table(K, E):
    def diag(M, k):
        return (M * jnp.eye(NODES, k=k, dtype=jnp.float32)).sum(axis=1)

    kc = diag(K, 0) + VOL_A * (-1.0)
    kl = diag(K, -1)
    kr = diag(K, 1)
    kd = diag(K, -13)
    ku = diag(K, 13)
    be = BOLTZ * diag(E, 0)
    t = jnp.stack([kc, kl, kr, kd, ku, be], axis=1)
    return jnp.repeat(t, 16, axis=1)


@functools.lru_cache(maxsize=None)
def _build_sc(B):
    assert B == NW * NCT * CT
    mesh = plsc.VectorSubcoreMesh(core_axis_name="c", subcore_axis_name="s")
    units = [(ct, nh) for ct in range(NCT) for nh in range(3)]

    @functools.partial(
        pl.kernel,
        mesh=mesh,
        out_type=jax.ShapeDtypeStruct((NW, 16), jnp.float32),
        compiler_params=pltpu.CompilerParams(use_tc_tiling_on_sc=True),
        scratch_types=(
            [pltpu.VMEM((TO_ROWS, CT), jnp.float32),
             pltpu.VMEM((X_ROWS, CT), jnp.float32),
             pltpu.VMEM((X_ROWS, CT), jnp.float32),
             pltpu.VMEM((X_ROWS, CT), jnp.float32),
             pltpu.VMEM((X_ROWS, CT), jnp.float32)] * 2
            + [
                pltpu.VMEM((NODES, 96), jnp.float32),
                pltpu.VMEM((16,), jnp.float32),
                pltpu.SemaphoreType.DMA,
                pltpu.SemaphoreType.DMA,
            ]
        ),
    )
    def sc_loss(to_h, tn_h, ht_h, if_h, te_h, coef_h, out_h,
                x0, x1, x2, x3, x4, y0, y1, y2, y3, y4,
                cf, accv, semA, semB):
        wid = lax.axis_index("s") * 2 + lax.axis_index("c")
        srcs = (to_h, tn_h, ht_h, if_h, te_h)
        halves = ((x0, x1, x2, x3, x4), (y0, y1, y2, y3, y4))
        sems = (semA, semB)
        pltpu.sync_copy(coef_h, cf)
        colbase = wid * (NCT * CT)

        def copies(u):
            ct, nh = units[u]
            bufs = halves[u % 2]
            c0 = colbase + ct * CT
            out = [(srcs[0].at[pl.ds(H0S[nh], HRS[nh]), pl.ds(c0, CT)],
                    bufs[0].at[pl.ds(0, HRS[nh]), pl.ds(0, CT)])]
            nr = N1S[nh] - N0S[nh]
            for k in range(1, 5):
                out.append((srcs[k].at[pl.ds(N0S[nh], nr), pl.ds(c0, CT)],
                            bufs[k].at[pl.ds(0, nr), pl.ds(0, CT)]))
            return out

        def issue(u):
            for s, d in copies(u):
                pltpu.make_async_copy(s, d, sems[u % 2]).start()

        def drain(u):
            for s, d in copies(u):
                pltpu.make_async_copy(s, d, sems[u % 2]).wait()

        def compute(u, acc):
            ct, nh = units[u]
            gTo, gTn, gHt, gIf, gTe = halves[u % 2]
            n0, n1, h0 = N0S[nh], N1S[nh], H0S[nh]
            nr = n1 - n0
            hr = HRS[nh]
            dT = n0 - h0

            def node_body(i, a, _n0=n0, _dT=dT, _hr=hr, _nh=nh):
                kc = cf[i + _n0, pl.ds(0, 16)]
                kl = cf[i + _n0, pl.ds(16, 16)]
                kr = cf[i + _n0, pl.ds(32, 16)]
                kd = cf[i + _n0, pl.ds(48, 16)]
                ku = cf[i + _n0, pl.ds(64, 16)]
                be = cf[i + _n0, pl.ds(80, 16)]
                rT = i + _dT
                rl = jnp.maximum(rT - 1, 0) if _nh == 0 else rT - 1
                rd = jnp.maximum(rT - 13, 0) if _nh == 0 else rT - 13
                rr = jnp.minimum(rT + 1, _hr - 1) if _nh == 2 else rT + 1
                ru = jnp.minimum(rT + 13, _hr - 1) if _nh == 2 else rT + 13
                for j in range(CT // 16):
                    cb = 16 * j
                    toc = gTo[rT, pl.ds(cb, 16)]
                    tol = gTo[rl, pl.ds(cb, 16)]
                    tor = gTo[rr, pl.ds(cb, 16)]
                    tod = gTo[rd, pl.ds(cb, 16)]
                    tou = gTo[ru, pl.ds(cb, 16)]
                    tnn = gTn[i, pl.ds(cb, 16)]
                    q1 = gHt[i, pl.ds(cb, 16)]
                    q2 = gIf[i, pl.ds(cb, 16)]
                    tee = gTe[i, pl.ds(cb, 16)]
                    t = VOL_A * tnn + kc * toc
                    t = t + kl * tol + kr * tor
                    t = t + kd * tod + ku * tou
                    t = t - q1 - q2
                    to2 = toc * toc
                    te2 = tee * tee
                    t = t + be * (to2 * to2 - te2 * te2)
                    a = a + jnp.abs(t)
                return a

            return lax.fori_loop(0, nr, node_body, acc)

        acc = jnp.zeros((16,), jnp.float32)
        issue(0)
        issue(1)
        for u in range(len(units)):
            drain(u)
            if u + 2 < len(units):
                issue(u + 2)
            acc = compute(u, acc)
        accv[...] = acc
        pltpu.sync_copy(accv, out_h.at[wid])

    return sc_loss


def kernel(T_new, T_old, heaters_input, interfaces_input, Tenv, K, E):
    B = T_new.shape[0]
    coef = _coef_table(K, E)
    sc = _build_sc(B)
    partials = sc(T_old.T, T_new.T,
                  heaters_input.astype(jnp.float32).T,
                  interfaces_input.astype(jnp.float32).T,
                  Tenv.T, coef)
    return jnp.sum(partials) / (B * NODES)


# speedup vs baseline: 3.1771x; 1.3046x over previous
import functools

import jax
import jax.numpy as jnp
from jax import lax
from jax.experimental import pallas as pl
from jax.experimental.pallas import tpu as pltpu
from jax.experimental.pallas import tpu_sc as plsc

NX = 13
NODES = NX * NX
L_SIZE = 0.1
THICKNESS = 0.001
RHO = 2700.0
CP = 900.0
DT = 1.0
DX = L_SIZE / (NX - 1)
DY = L_SIZE / (NX - 1)
BOLTZ = 5.67e-08
VOL_A = RHO * CP * THICKNESS * DX * DY / DT

NW = 32
CT = 128
NCT = 2
TC_BN = 512
N0S = (0, 56, 112)
N1S = (56, 112, NODES)
H0S = (0, 40, 96)
HRS = (72, 88, NODES - 96)
TO_ROWS = 88
X_ROWS = 64


def _coef_table(K, E):
    def diag(M, k):
        return (M * jnp.eye(NODES, k=k, dtype=jnp.float32)).sum(axis=1)

    kc = diag(K, 0) + VOL_A * (-1.0)
    kl = diag(K, -1)
    kr = diag(K, 1)
    kd = diag(K, -13)
    ku = diag(K, 13)
    be = BOLTZ * diag(E, 0)
    t = jnp.stack([kc, kl, kr, kd, ku, be], axis=1)
    return jnp.repeat(t, 16, axis=1)


@functools.lru_cache(maxsize=None)
def _build_tc(n_blocks, off_blocks):
    def tc_body(to_ref, tn_ref, ht_ref, if_ref, te_ref, k_ref, be_ref,
                out_ref):
        i = pl.program_id(0)
        to = to_ref[...]
        kto = jax.lax.dot_general(
            k_ref[...], to, (((1,), (0,)), ((), ())),
            precision=jax.lax.Precision.HIGHEST,
            preferred_element_type=jnp.float32)
        be = be_ref[...][:, 0:1]
        to2 = to * to
        te = te_ref[...]
        te2 = te * te
        res = VOL_A * (tn_ref[...] - to) + kto
        res = res - ht_ref[...] - if_ref[...]
        res = res + be * (to2 * to2 - te2 * te2)
        part = jnp.sum(jnp.abs(res))

        @pl.when(i == 0)
        def _():
            out_ref[0, 0] = 0.0

        out_ref[0, 0] += part

    grid = (n_blocks,)
    return pl.pallas_call(
        tc_body,
        grid=grid,
        in_specs=[
            pl.BlockSpec((NODES, TC_BN), lambda i: (0, off_blocks + i)),
            pl.BlockSpec((NODES, TC_BN), lambda i: (0, off_blocks + i)),
            pl.BlockSpec((NODES, TC_BN), lambda i: (0, off_blocks + i)),
            pl.BlockSpec((NODES, TC_BN), lambda i: (0, off_blocks + i)),
            pl.BlockSpec((NODES, TC_BN), lambda i: (0, off_blocks + i)),
            pl.BlockSpec((NODES, NODES), lambda i: (0, 0)),
            pl.BlockSpec((NODES, 128), lambda i: (0, 0)),
        ],
        out_specs=pl.BlockSpec((1, 1), lambda i: (0, 0),
                               memory_space=pltpu.SMEM),
        out_shape=jax.ShapeDtypeStruct((1, 1), jnp.float32),
    )


@functools.lru_cache(maxsize=None)
def _build_sc(B):
    assert B == NW * NCT * CT
    mesh = plsc.VectorSubcoreMesh(core_axis_name="c", subcore_axis_name="s")
    units = [(ct, nh) for ct in range(NCT) for nh in range(3)]

    @functools.partial(
        pl.kernel,
        mesh=mesh,
        out_type=jax.ShapeDtypeStruct((NW, 16), jnp.float32),
        compiler_params=pltpu.CompilerParams(use_tc_tiling_on_sc=True),
        scratch_types=(
            [pltpu.VMEM((TO_ROWS, CT), jnp.float32),
             pltpu.VMEM((X_ROWS, CT), jnp.float32),
             pltpu.VMEM((X_ROWS, CT), jnp.float32),
             pltpu.VMEM((X_ROWS, CT), jnp.float32),
             pltpu.VMEM((X_ROWS, CT), jnp.float32)] * 2
            + [
                pltpu.VMEM((NODES, 96), jnp.float32),
                pltpu.VMEM((16,), jnp.float32),
                pltpu.SemaphoreType.DMA,
                pltpu.SemaphoreType.DMA,
            ]
        ),
    )
    def sc_loss(to_h, tn_h, ht_h, if_h, te_h, coef_h, out_h,
                x0, x1, x2, x3, x4, y0, y1, y2, y3, y4,
                cf, accv, semA, semB):
        wid = lax.axis_index("s") * 2 + lax.axis_index("c")
        srcs = (to_h, tn_h, ht_h, if_h, te_h)
        halves = ((x0, x1, x2, x3, x4), (y0, y1, y2, y3, y4))
        sems = (semA, semB)
        pltpu.sync_copy(coef_h, cf)
        colbase = wid * (NCT * CT)

        def copies(u):
            ct, nh = units[u]
            bufs = halves[u % 2]
            c0 = colbase + ct * CT
            out = [(srcs[0].at[pl.ds(H0S[nh], HRS[nh]), pl.ds(c0, CT)],
                    bufs[0].at[pl.ds(0, HRS[nh]), pl.ds(0, CT)])]
            nr = N1S[nh] - N0S[nh]
            for k in range(1, 5):
                out.append((srcs[k].at[pl.ds(N0S[nh], nr), pl.ds(c0, CT)],
                            bufs[k].at[pl.ds(0, nr), pl.ds(0, CT)]))
            return out

        def issue(u):
            for s, d in copies(u):
                pltpu.make_async_copy(s, d, sems[u % 2]).start()

        def drain(u):
            for s, d in copies(u):
                pltpu.make_async_copy(s, d, sems[u % 2]).wait()

        def compute(u, acc):
            ct, nh = units[u]
            gTo, gTn, gHt, gIf, gTe = halves[u % 2]
            n0, n1, h0 = N0S[nh], N1S[nh], H0S[nh]
            nr = n1 - n0
            hr = HRS[nh]
            dT = n0 - h0

            def node_body(i, a, _n0=n0, _dT=dT, _hr=hr, _nh=nh):
                kc = cf[i + _n0, pl.ds(0, 16)]
                kl = cf[i + _n0, pl.ds(16, 16)]
                kr = cf[i + _n0, pl.ds(32, 16)]
                kd = cf[i + _n0, pl.ds(48, 16)]
                ku = cf[i + _n0, pl.ds(64, 16)]
                be = cf[i + _n0, pl.ds(80, 16)]
                rT = i + _dT
                rl = jnp.maximum(rT - 1, 0) if _nh == 0 else rT - 1
                rd = jnp.maximum(rT - 13, 0) if _nh == 0 else rT - 13
                rr = jnp.minimum(rT + 1, _hr - 1) if _nh == 2 else rT + 1
                ru = jnp.minimum(rT + 13, _hr - 1) if _nh == 2 else rT + 13
                for j in range(CT // 16):
                    cb = 16 * j
                    toc = gTo[rT, pl.ds(cb, 16)]
                    tol = gTo[rl, pl.ds(cb, 16)]
                    tor = gTo[rr, pl.ds(cb, 16)]
                    tod = gTo[rd, pl.ds(cb, 16)]
                    tou = gTo[ru, pl.ds(cb, 16)]
                    tnn = gTn[i, pl.ds(cb, 16)]
                    q1 = gHt[i, pl.ds(cb, 16)]
                    q2 = gIf[i, pl.ds(cb, 16)]
                    tee = gTe[i, pl.ds(cb, 16)]
                    t = VOL_A * tnn + kc * toc
                    t = t + kl * tol + kr * tor
                    t = t + kd * tod + ku * tou
                    t = t - q1 - q2
                    to2 = toc * toc
                    te2 = tee * tee
                    t = t + be * (to2 * to2 - te2 * te2)
                    a = a + jnp.abs(t)
                return a

            return lax.fori_loop(0, nr, node_body, acc)

        acc = jnp.zeros((16,), jnp.float32)
        issue(0)
        issue(1)
        for u in range(len(units)):
            drain(u)
            if u + 2 < len(units):
                issue(u + 2)
            acc = compute(u, acc)
        accv[...] = acc
        pltpu.sync_copy(accv, out_h.at[wid])

    return sc_loss


def kernel(T_new, T_old, heaters_input, interfaces_input, Tenv, K, E):
    B = T_new.shape[0]
    coef = _coef_table(K, E)
    b_sc = NW * NCT * CT
    sc = _build_sc(b_sc)
    tT_old = T_old.T
    tT_new = T_new.T
    tHt = heaters_input.astype(jnp.float32).T
    tIf = interfaces_input.astype(jnp.float32).T
    tTe = Tenv.T
    partials = sc(tT_old, tT_new, tHt, tIf, tTe, coef)
    be_rep = jnp.repeat(
        (BOLTZ * (E * jnp.eye(NODES, dtype=jnp.float32)).sum(axis=1))[:, None],
        128, axis=1)
    tc = _build_tc((B - b_sc) // TC_BN, b_sc // TC_BN)
    tc_sum = tc(tT_old, tT_new, tHt, tIf, tTe, K, be_rep)
    return (jnp.sum(partials) + tc_sum[0, 0]) / (B * NODES)
